# baseline (speedup 1.0000x reference)
import functools

import jax
import jax.numpy as jnp
from jax import lax
from jax.experimental import pallas as pl
from jax.experimental.pallas import tpu as pltpu
from jax.experimental.pallas import tpu_sc as plsc

B = 16384
CONT = 13
NF = 26
V = 100000
D = 16

NC = 2
NS = 16
NW = NC * NS
ROWS_W = B // NW
RP = B // 128
WP = ROWS_W // 128
CB = 64
NCHUNK = ROWS_W // CB
OUT_W = CONT + NF * D


def _sc_kernel(cate_hbm, cont_hbm, table_hbm, out_hbm,
               fcate_v, em_v, cont_v, row_v, gsem):
    wid = lax.axis_index("s") * NC + lax.axis_index("c")

    for f in range(NF):
        pltpu.sync_copy(cate_hbm.at[pl.ds(f * RP + wid * WP, WP)],
                        fcate_v.at[f])
    def fix_body(j, carry):
        p = j // (128 // 16)
        k = lax.rem(j, 128 // 16)
        s = pl.ds(k * 16, 16)
        for f in range(NF):
            fcate_v[f, p, s] = fcate_v[f, p, s] + f * V
        return carry

    lax.fori_loop(0, WP * (128 // 16), fix_body, 0)

    def fire(c, buf):
        p = c // 2
        s = pl.ds(lax.rem(c, 2) * CB, CB)
        for f in range(NF):
            pltpu.async_copy(table_hbm.at[fcate_v.at[f, p, s]],
                             em_v.at[buf, f], gsem)

    def drain(c, buf):
        p = c // 2
        s = pl.ds(lax.rem(c, 2) * CB, CB)
        for f in range(NF):
            pltpu.make_async_copy(table_hbm.at[fcate_v.at[f, p, s]],
                                  em_v.at[buf, f], gsem).wait()

    fire(0, 0)

    def chunk_body(c, carry):
        buf = lax.rem(c, 2)
        row0 = wid * ROWS_W + c * CB
        pltpu.sync_copy(cont_hbm.at[pl.ds(row0, CB)], cont_v)
        drain(c, buf)

        @pl.when(c + 1 < NCHUNK)
        def _():
            fire(c + 1, lax.rem(c + 1, 2))

        def assemble(b, cc):
            row_v[b, pl.ds(0, 16)] = cont_v[b]
            for f in range(NF):
                row_v[b, pl.ds(CONT + f * D, D)] = em_v[buf, f, b]
            return cc

        lax.fori_loop(0, CB, assemble, c)
        pltpu.sync_copy(row_v, out_hbm.at[pl.ds(row0, CB)])
        return carry

    lax.fori_loop(0, NCHUNK, chunk_body, 0)


@jax.jit
def kernel(x_cont, x_cate, tables):
    cate_t = x_cate.T.reshape(NF * RP, 128)
    cont_pad = jnp.pad(x_cont, ((0, 0), (0, 3)))
    table_flat = tables.reshape(NF * V, D)
    mesh = plsc.VectorSubcoreMesh(core_axis_name="c", subcore_axis_name="s")
    run = functools.partial(
        pl.kernel,
        mesh=mesh,
        compiler_params=pltpu.CompilerParams(use_tc_tiling_on_sc=False),
        out_type=jax.ShapeDtypeStruct((B, OUT_W), jnp.float32),
        scratch_types=[
            pltpu.VMEM((NF, WP, 128), jnp.int32),
            pltpu.VMEM((2, NF, CB, D), jnp.float32),
            pltpu.VMEM((CB, 16), jnp.float32),
            pltpu.VMEM((CB, OUT_W), jnp.float32),
            pltpu.SemaphoreType.DMA,
        ],
    )(_sc_kernel)
    return run(cate_t, cont_pad, table_flat)
